# baseline (speedup 1.0000x reference)
import functools

import jax
import jax.numpy as jnp
from jax import lax
from jax.experimental import pallas as pl
from jax.experimental.pallas import tpu as pltpu
from jax.experimental.pallas import tpu_sc as plsc

N = 50000
NPAD = 50176
NROWS = 392
M = 1250
MPAD = 1280
K = 20
CIN = 128
COUT = 128
PADV = 1.0e6
BIG = 2 ** 30


def _fps_body(px_ref, py_ref, pz_ref, pt_ref, idx_ref, sp_ref, dists_ref,
              iota_ref):
    flat = (lax.broadcasted_iota(jnp.int32, (NROWS, 128), 0) * 128
            + lax.broadcasted_iota(jnp.int32, (NROWS, 128), 1))
    iota_ref[...] = flat
    dists_ref[...] = jnp.where(flat < N, jnp.float32(1e10), jnp.float32(-1e30))
    idx_ref[...] = jnp.zeros((MPAD, 8), jnp.int32)
    sp_ref[...] = jnp.zeros((MPAD, 8), jnp.float32)
    prow0 = pt_ref[0:1, :]
    sp_ref[0:1, :] = prow0

    def body(i, prow):
        lx = prow[0:1, 0:1]
        ly = prow[0:1, 1:2]
        lz = prow[0:1, 2:3]
        d = ((px_ref[...] - lx) ** 2 + (py_ref[...] - ly) ** 2
             + (pz_ref[...] - lz) ** 2)
        nd = jnp.minimum(dists_ref[...], d)
        dists_ref[...] = nd
        m = jnp.max(nd)
        idx = jnp.min(jnp.where(nd == m, iota_ref[...], BIG))
        idx_ref[pl.ds(i, 1), :] = jnp.full((1, 8), idx, jnp.int32)
        prow_new = pt_ref[pl.ds(idx, 1), :]
        sp_ref[pl.ds(i, 1), :] = prow_new
        return prow_new

    lax.fori_loop(1, M, body, prow0)


def _run_fps(px, py, pz, pt):
    return pl.pallas_call(
        _fps_body,
        out_shape=[
            jax.ShapeDtypeStruct((MPAD, 8), jnp.int32),
            jax.ShapeDtypeStruct((MPAD, 8), jnp.float32),
        ],
        scratch_shapes=[
            pltpu.VMEM((NROWS, 128), jnp.float32),
            pltpu.VMEM((NROWS, 128), jnp.int32),
        ],
    )(px, py, pz, pt)


NGRP = NPAD // 128
CB = 64
INFF = float("inf")


def _knn_body(sp_ref, px_ref, py_ref, pz_ref, out_ref):
    sx = sp_ref[:, :, 0:1]
    sy = sp_ref[:, :, 1:2]
    sz = sp_ref[:, :, 2:3]

    def dist():
        return ((sx - px_ref[...]) ** 2 + (sy - py_ref[...]) ** 2
                + (sz - pz_ref[...]) ** 2)

    jio = lax.broadcasted_iota(jnp.int32, (CB, 128, NGRP), 2)

    def argmin3(c, m):
        return jnp.min(jnp.where(c == m[:, :, None], jio, BIG), axis=2)

    d2 = dist()
    m1 = jnp.min(d2, axis=2)
    a1 = argmin3(d2, m1)
    c2 = jnp.where(jio == a1[:, :, None], INFF, d2)
    m2 = jnp.min(c2, axis=2)
    a2 = argmin3(c2, m2)
    c3 = jnp.where(jio == a2[:, :, None], INFF, c2)
    m3 = jnp.min(c3, axis=2)
    a3 = argmin3(c3, m3)

    liota = lax.broadcasted_iota(jnp.int32, (CB, 128), 1)

    def ext(k, st):
        idxs, m1, a1, m2, a2, m3, a3, thv, thj = st
        mrow = jnp.min(m1, axis=1, keepdims=True)
        lane = jnp.min(jnp.where(m1 == mrow, liota, BIG), axis=1,
                       keepdims=True)
        sel = liota == lane
        jsel = jnp.min(jnp.where(sel, a1, BIG), axis=1, keepdims=True)
        idxs = jnp.where(liota == k, lane * NGRP + jsel, idxs)
        thv = jnp.where(sel, mrow, thv)
        thj = jnp.where(sel, jsel, thj)
        m1 = jnp.where(sel, m2, m1)
        a1 = jnp.where(sel, a2, a1)
        m2 = jnp.where(sel, m3, m2)
        a2 = jnp.where(sel, a3, a2)
        m3 = jnp.where(sel, INFF, m3)
        a3 = jnp.where(sel, BIG, a3)
        need = sel & (m1 == INFF)

        def refill(op):
            q1, b1, q2, b2, q3, b3 = op
            dv = dist()
            tv = jnp.where(need, thv, INFF)
            tj = jnp.where(need, thj, BIG)
            e0 = ((dv > tv[:, :, None])
                  | ((dv == tv[:, :, None]) & (jio > tj[:, :, None])))
            r1 = jnp.where(e0, dv, INFF)
            h1 = jnp.min(r1, axis=2)
            f1 = argmin3(r1, h1)
            r2 = jnp.where(jio == f1[:, :, None], INFF, r1)
            h2 = jnp.min(r2, axis=2)
            f2 = argmin3(r2, h2)
            r3 = jnp.where(jio == f2[:, :, None], INFF, r2)
            h3 = jnp.min(r3, axis=2)
            f3 = argmin3(r3, h3)
            return (jnp.where(need, h1, q1), jnp.where(need, f1, b1),
                    jnp.where(need, h2, q2), jnp.where(need, f2, b2),
                    jnp.where(need, h3, q3), jnp.where(need, f3, b3))

        m1, a1, m2, a2, m3, a3 = lax.cond(
            jnp.any(need), refill, lambda op: op, (m1, a1, m2, a2, m3, a3))
        return (idxs, m1, a1, m2, a2, m3, a3, thv, thj)

    st0 = (jnp.zeros((CB, 128), jnp.int32), m1, a1, m2, a2, m3, a3,
           jnp.zeros((CB, 128), jnp.float32), jnp.zeros((CB, 128), jnp.int32))
    out_ref[...] = lax.fori_loop(0, K, ext, st0)[0]


def _run_knn(subpos3, px3, py3, pz3):
    nblk = MPAD // CB
    return pl.pallas_call(
        _knn_body,
        grid=(nblk,),
        in_specs=[
            pl.BlockSpec((CB, 1, 8), lambda i: (i, 0, 0)),
            pl.BlockSpec((1, 128, NGRP), lambda i: (0, 0, 0)),
            pl.BlockSpec((1, 128, NGRP), lambda i: (0, 0, 0)),
            pl.BlockSpec((1, 128, NGRP), lambda i: (0, 0, 0)),
        ],
        out_specs=pl.BlockSpec((CB, 128), lambda i: (i, 0)),
        out_shape=jax.ShapeDtypeStruct((MPAD, 128), jnp.int32),
    )(subpos3, px3, py3, pz3)


def _mm_body(x_ref, w_ref, b_out_ref, c1_out_ref):
    xb = x_ref[...]
    w1 = w_ref[:, 0:CIN]
    w2 = w_ref[:, CIN:2 * CIN]
    dn = (((1,), (1,)), ((), ()))
    b_out_ref[...] = lax.dot_general(xb, w2, dn,
                                     preferred_element_type=jnp.float32)
    c1_out_ref[...] = lax.dot_general(xb, w1 - w2, dn,
                                      preferred_element_type=jnp.float32)


def _run_mm(xfp, w):
    nblk = NPAD // 128
    return pl.pallas_call(
        _mm_body,
        grid=(nblk,),
        in_specs=[
            pl.BlockSpec((128, CIN), lambda i: (i, 0)),
            pl.BlockSpec((COUT, 2 * CIN), lambda i: (0, 0)),
        ],
        out_specs=[
            pl.BlockSpec((128, COUT), lambda i: (i, 0)),
            pl.BlockSpec((128, COUT), lambda i: (i, 0)),
        ],
        out_shape=[
            jax.ShapeDtypeStruct((NPAD, COUT), jnp.float32),
            jax.ShapeDtypeStruct((NPAD, COUT), jnp.float32),
        ],
    )(xfp, w)


NEDGE = MPAD * K


def _run_sc_gather(bmat, c1mat, nn_flat, fps_flat):
    info = plsc.get_sparse_core_info()
    nc, ns = info.num_cores, info.num_subcores
    nw = nc * ns
    e_per_w = NEDGE // nw
    m_per_w = MPAD // nw
    mesh = plsc.VectorSubcoreMesh(core_axis_name="c", subcore_axis_name="s")

    @functools.partial(
        pl.kernel,
        mesh=mesh,
        out_type=[
            jax.ShapeDtypeStruct((NEDGE, COUT), jnp.float32),
            jax.ShapeDtypeStruct((MPAD, COUT), jnp.float32),
        ],
        scratch_types=[
            pltpu.VMEM((e_per_w,), jnp.int32),
            pltpu.VMEM((e_per_w, COUT), jnp.float32),
            pltpu.VMEM((m_per_w,), jnp.int32),
            pltpu.VMEM((m_per_w, COUT), jnp.float32),
            pltpu.SemaphoreType.DMA,
        ],
    )
    def k(b_hbm, c1_hbm, nn_hbm, fps_hbm, out1_hbm, out2_hbm,
          idx1_v, rows1_v, idx2_v, rows2_v, sem):
        wid = lax.axis_index("s") * nc + lax.axis_index("c")
        base1 = wid * e_per_w
        pltpu.sync_copy(nn_hbm.at[pl.ds(base1, e_per_w)], idx1_v)
        pltpu.async_copy(b_hbm.at[idx1_v], rows1_v, sem).wait()
        pltpu.sync_copy(rows1_v, out1_hbm.at[pl.ds(base1, e_per_w)])
        base2 = wid * m_per_w
        pltpu.sync_copy(fps_hbm.at[pl.ds(base2, m_per_w)], idx2_v)
        pltpu.async_copy(c1_hbm.at[idx2_v], rows2_v, sem).wait()
        pltpu.sync_copy(rows2_v, out2_hbm.at[pl.ds(base2, m_per_w)])

    return k(bmat, c1mat, nn_flat, fps_flat)


MBLK = 128
NEBLK = MPAD // MBLK


def _edge_body(xj_ref, c1_ref, b_ref, gmax_ref, gmin_ref, s_ref, sq_ref,
               acc_s, acc_q):
    pid = pl.program_id(0)

    @pl.when(pid == 0)
    def _():
        acc_s[...] = jnp.zeros((8, 128), jnp.float32)
        acc_q[...] = jnp.zeros((8, 128), jnp.float32)

    bvec = b_ref[...]

    def body(r, _):
        h = jnp.maximum(
            xj_ref[pl.ds(r * K, K), :] + c1_ref[pl.ds(r, 1), :] + bvec, 0.0)
        gmax_ref[pl.ds(r, 1), :] = jnp.max(h, axis=0, keepdims=True)
        gmin_ref[pl.ds(r, 1), :] = jnp.min(h, axis=0, keepdims=True)
        valid = (pid * MBLK + r) < M
        s1 = jnp.sum(h, axis=0, keepdims=True)
        q1 = jnp.sum(h * h, axis=0, keepdims=True)
        zero = jnp.zeros((1, 128), jnp.float32)
        acc_s[0:1, :] += jnp.where(valid, s1, zero)
        acc_q[0:1, :] += jnp.where(valid, q1, zero)
        return 0

    lax.fori_loop(0, MBLK, body, 0)

    @pl.when(pid == NEBLK - 1)
    def _():
        s_ref[...] = acc_s[...]
        sq_ref[...] = acc_q[...]


def _run_edge(xjb, subc1, brow):
    return pl.pallas_call(
        _edge_body,
        grid=(NEBLK,),
        in_specs=[
            pl.BlockSpec((MBLK * K, COUT), lambda i: (i, 0)),
            pl.BlockSpec((MBLK, COUT), lambda i: (i, 0)),
            pl.BlockSpec((1, COUT), lambda i: (0, 0)),
        ],
        out_specs=[
            pl.BlockSpec((MBLK, COUT), lambda i: (i, 0)),
            pl.BlockSpec((MBLK, COUT), lambda i: (i, 0)),
            pl.BlockSpec((8, COUT), lambda i: (0, 0)),
            pl.BlockSpec((8, COUT), lambda i: (0, 0)),
        ],
        out_shape=[
            jax.ShapeDtypeStruct((MPAD, COUT), jnp.float32),
            jax.ShapeDtypeStruct((MPAD, COUT), jnp.float32),
            jax.ShapeDtypeStruct((8, COUT), jnp.float32),
            jax.ShapeDtypeStruct((8, COUT), jnp.float32),
        ],
        scratch_shapes=[
            pltpu.VMEM((8, 128), jnp.float32),
            pltpu.VMEM((8, 128), jnp.float32),
        ],
    )(xjb, subc1, brow)


def _norm_body(gmax_ref, gmin_ref, s_ref, sq_ref, gamma_ref, beta_ref,
               out_ref):
    cnt = jnp.float32(M * K)
    mean = s_ref[0:1, :] / cnt
    var = sq_ref[0:1, :] / cnt - mean * mean
    sd = jnp.sqrt(var + jnp.float32(1e-5))
    gamma = gamma_ref[...]
    beta = beta_ref[...]
    g = jnp.where(gamma >= 0, gmax_ref[...], gmin_ref[...])
    out_ref[...] = (g - mean) / sd * gamma + beta


def _run_norm(gmax, gmin, s, sq, gamma, beta):
    return pl.pallas_call(
        _norm_body,
        out_shape=jax.ShapeDtypeStruct((MPAD, COUT), jnp.float32),
    )(gmax, gmin, s, sq, gamma, beta)


def kernel(x, pos, W, b, gamma, beta):
    p = pos[0, :, :, 0]
    xf = x[0, :, :, 0].T

    pad = NPAD - N
    pxr = jnp.pad(p[0:1], ((0, 0), (0, pad)), constant_values=PADV)
    pyr = jnp.pad(p[1:2], ((0, 0), (0, pad)), constant_values=PADV)
    pzr = jnp.pad(p[2:3], ((0, 0), (0, pad)), constant_values=PADV)
    px = pxr.reshape(NROWS, 128)
    py = pyr.reshape(NROWS, 128)
    pz = pzr.reshape(NROWS, 128)
    pt = jnp.pad(p.T, ((0, pad), (0, 5)), constant_values=0.0)
    pt = jnp.where(
        (jnp.arange(NPAD)[:, None] < N) | (jnp.arange(8)[None, :] >= 3),
        pt, PADV)

    fpsidx, subpos = _run_fps(px, py, pz, pt)
    nnidx = _run_knn(subpos.reshape(MPAD, 1, 8), pxr.reshape(1, 128, NGRP),
                     pyr.reshape(1, 128, NGRP), pzr.reshape(1, 128, NGRP))

    xfp = jnp.pad(xf, ((0, pad), (0, 0)))
    bmat, c1mat = _run_mm(xfp, W)

    nn_flat = nnidx[:, :K].reshape(-1)
    fps_flat = fpsidx[:, 0]
    xjb, subc1 = _run_sc_gather(bmat, c1mat, nn_flat, fps_flat)

    gmax, gmin, s, sq = _run_edge(xjb, subc1, b.reshape(1, COUT))
    out = _run_norm(gmax, gmin, s, sq, gamma.reshape(1, COUT),
                    beta.reshape(1, COUT))
    return out[:M].T[None, :, :, None]


# speedup vs baseline: 1.3634x; 1.3634x over previous
import functools

import jax
import jax.numpy as jnp
from jax import lax
from jax.experimental import pallas as pl
from jax.experimental.pallas import tpu as pltpu
from jax.experimental.pallas import tpu_sc as plsc

N = 50000
NPAD = 50176
NROWS = 392
M = 1250
MPAD = 1280
K = 20
CIN = 128
COUT = 128
PADV = 1.0e6
BIG = 2 ** 30


def _fps_body(px_ref, py_ref, pz_ref, pt_ref, idx_ref, sp_ref, dists_ref,
              iota_ref):
    flat = (lax.broadcasted_iota(jnp.int32, (NROWS, 128), 0) * 128
            + lax.broadcasted_iota(jnp.int32, (NROWS, 128), 1))
    iota_ref[...] = flat
    dists_ref[...] = jnp.where(flat < N, jnp.float32(1e10), jnp.float32(-1e30))
    idx_ref[...] = jnp.zeros((MPAD, 8), jnp.int32)
    sp_ref[...] = jnp.zeros((MPAD, 8), jnp.float32)
    prow0 = pt_ref[0:1, :]
    sp_ref[0:1, :] = prow0

    def body(i, prow):
        lx = prow[0:1, 0:1]
        ly = prow[0:1, 1:2]
        lz = prow[0:1, 2:3]
        d = ((px_ref[...] - lx) ** 2 + (py_ref[...] - ly) ** 2
             + (pz_ref[...] - lz) ** 2)
        nd = jnp.minimum(dists_ref[...], d)
        dists_ref[...] = nd
        m = jnp.max(nd)
        idx = jnp.min(jnp.where(nd == m, iota_ref[...], BIG))
        idx_ref[pl.ds(i, 1), :] = jnp.full((1, 8), idx, jnp.int32)
        prow_new = pt_ref[pl.ds(idx, 1), :]
        sp_ref[pl.ds(i, 1), :] = prow_new
        return prow_new

    lax.fori_loop(1, M, body, prow0)


def _run_fps(px, py, pz, pt):
    return pl.pallas_call(
        _fps_body,
        out_shape=[
            jax.ShapeDtypeStruct((MPAD, 8), jnp.int32),
            jax.ShapeDtypeStruct((MPAD, 8), jnp.float32),
        ],
        scratch_shapes=[
            pltpu.VMEM((NROWS, 128), jnp.float32),
            pltpu.VMEM((NROWS, 128), jnp.int32),
        ],
    )(px, py, pz, pt)


NGRP = NPAD // 128
CB = 32
INFF = float("inf")


def _knn_body(sp_ref, px_ref, py_ref, pz_ref, out_ref):
    sx = sp_ref[:, :, 0:1]
    sy = sp_ref[:, :, 1:2]
    sz = sp_ref[:, :, 2:3]

    def dist():
        return ((sx - px_ref[...]) ** 2 + (sy - py_ref[...]) ** 2
                + (sz - pz_ref[...]) ** 2)

    jio = lax.broadcasted_iota(jnp.int32, (CB, 128, NGRP), 2)

    def argmin3(c, m):
        return jnp.min(jnp.where(c == m[:, :, None], jio, BIG), axis=2)

    d2 = dist()
    m1 = jnp.min(d2, axis=2)
    a1 = argmin3(d2, m1)
    c2 = jnp.where(jio == a1[:, :, None], INFF, d2)
    m2 = jnp.min(c2, axis=2)
    a2 = argmin3(c2, m2)
    c3 = jnp.where(jio == a2[:, :, None], INFF, c2)
    m3 = jnp.min(c3, axis=2)
    a3 = argmin3(c3, m3)

    liota = lax.broadcasted_iota(jnp.int32, (CB, 128), 1)

    def ext(k, st):
        idxs, m1, a1, m2, a2, m3, a3, thv, thj = st
        mrow = jnp.min(m1, axis=1, keepdims=True)
        lane = jnp.min(jnp.where(m1 == mrow, liota, BIG), axis=1,
                       keepdims=True)
        sel = liota == lane
        jsel = jnp.min(jnp.where(sel, a1, BIG), axis=1, keepdims=True)
        idxs = jnp.where(liota == k, lane * NGRP + jsel, idxs)
        thv = jnp.where(sel, mrow, thv)
        thj = jnp.where(sel, jsel, thj)
        m1 = jnp.where(sel, m2, m1)
        a1 = jnp.where(sel, a2, a1)
        m2 = jnp.where(sel, m3, m2)
        a2 = jnp.where(sel, a3, a2)
        m3 = jnp.where(sel, INFF, m3)
        a3 = jnp.where(sel, BIG, a3)
        need = sel & (m1 == INFF)

        def refill(op):
            q1, b1, q2, b2, q3, b3 = op
            dv = dist()
            tv = jnp.where(need, thv, INFF)
            tj = jnp.where(need, thj, BIG)
            e0 = ((dv > tv[:, :, None])
                  | ((dv == tv[:, :, None]) & (jio > tj[:, :, None])))
            r1 = jnp.where(e0, dv, INFF)
            h1 = jnp.min(r1, axis=2)
            f1 = argmin3(r1, h1)
            r2 = jnp.where(jio == f1[:, :, None], INFF, r1)
            h2 = jnp.min(r2, axis=2)
            f2 = argmin3(r2, h2)
            r3 = jnp.where(jio == f2[:, :, None], INFF, r2)
            h3 = jnp.min(r3, axis=2)
            f3 = argmin3(r3, h3)
            return (jnp.where(need, h1, q1), jnp.where(need, f1, b1),
                    jnp.where(need, h2, q2), jnp.where(need, f2, b2),
                    jnp.where(need, h3, q3), jnp.where(need, f3, b3))

        m1, a1, m2, a2, m3, a3 = lax.cond(
            jnp.any(need), refill, lambda op: op, (m1, a1, m2, a2, m3, a3))
        return (idxs, m1, a1, m2, a2, m3, a3, thv, thj)

    st0 = (jnp.zeros((CB, 128), jnp.int32), m1, a1, m2, a2, m3, a3,
           jnp.zeros((CB, 128), jnp.float32), jnp.zeros((CB, 128), jnp.int32))
    out_ref[...] = lax.fori_loop(0, K, ext, st0)[0]


def _run_knn(subpos3, px3, py3, pz3):
    nblk = MPAD // CB
    return pl.pallas_call(
        _knn_body,
        grid=(nblk,),
        in_specs=[
            pl.BlockSpec((CB, 1, 8), lambda i: (i, 0, 0)),
            pl.BlockSpec((1, 128, NGRP), lambda i: (0, 0, 0)),
            pl.BlockSpec((1, 128, NGRP), lambda i: (0, 0, 0)),
            pl.BlockSpec((1, 128, NGRP), lambda i: (0, 0, 0)),
        ],
        out_specs=pl.BlockSpec((CB, 128), lambda i: (i, 0)),
        out_shape=jax.ShapeDtypeStruct((MPAD, 128), jnp.int32),
    )(subpos3, px3, py3, pz3)


def _mm_body(x_ref, w_ref, b_out_ref, c1_out_ref):
    xb = x_ref[...]
    w1 = w_ref[:, 0:CIN]
    w2 = w_ref[:, CIN:2 * CIN]
    dn = (((1,), (1,)), ((), ()))
    b_out_ref[...] = lax.dot_general(xb, w2, dn,
                                     preferred_element_type=jnp.float32)
    c1_out_ref[...] = lax.dot_general(xb, w1 - w2, dn,
                                      preferred_element_type=jnp.float32)


def _run_mm(xfp, w):
    nblk = NPAD // 128
    return pl.pallas_call(
        _mm_body,
        grid=(nblk,),
        in_specs=[
            pl.BlockSpec((128, CIN), lambda i: (i, 0)),
            pl.BlockSpec((COUT, 2 * CIN), lambda i: (0, 0)),
        ],
        out_specs=[
            pl.BlockSpec((128, COUT), lambda i: (i, 0)),
            pl.BlockSpec((128, COUT), lambda i: (i, 0)),
        ],
        out_shape=[
            jax.ShapeDtypeStruct((NPAD, COUT), jnp.float32),
            jax.ShapeDtypeStruct((NPAD, COUT), jnp.float32),
        ],
    )(xfp, w)


NEDGE = MPAD * K


def _run_sc_gather(bmat, c1mat, nn_flat, fps_flat):
    info = plsc.get_sparse_core_info()
    nc, ns = info.num_cores, info.num_subcores
    nw = nc * ns
    e_per_w = NEDGE // nw
    m_per_w = MPAD // nw
    mesh = plsc.VectorSubcoreMesh(core_axis_name="c", subcore_axis_name="s")

    @functools.partial(
        pl.kernel,
        mesh=mesh,
        out_type=[
            jax.ShapeDtypeStruct((NEDGE, COUT), jnp.float32),
            jax.ShapeDtypeStruct((MPAD, COUT), jnp.float32),
        ],
        scratch_types=[
            pltpu.VMEM((e_per_w,), jnp.int32),
            pltpu.VMEM((e_per_w, COUT), jnp.float32),
            pltpu.VMEM((m_per_w,), jnp.int32),
            pltpu.VMEM((m_per_w, COUT), jnp.float32),
            pltpu.SemaphoreType.DMA,
        ],
    )
    def k(b_hbm, c1_hbm, nn_hbm, fps_hbm, out1_hbm, out2_hbm,
          idx1_v, rows1_v, idx2_v, rows2_v, sem):
        wid = lax.axis_index("s") * nc + lax.axis_index("c")
        base1 = wid * e_per_w
        pltpu.sync_copy(nn_hbm.at[pl.ds(base1, e_per_w)], idx1_v)
        pltpu.async_copy(b_hbm.at[idx1_v], rows1_v, sem).wait()
        pltpu.sync_copy(rows1_v, out1_hbm.at[pl.ds(base1, e_per_w)])
        base2 = wid * m_per_w
        pltpu.sync_copy(fps_hbm.at[pl.ds(base2, m_per_w)], idx2_v)
        pltpu.async_copy(c1_hbm.at[idx2_v], rows2_v, sem).wait()
        pltpu.sync_copy(rows2_v, out2_hbm.at[pl.ds(base2, m_per_w)])

    return k(bmat, c1mat, nn_flat, fps_flat)


MBLK = 128
NEBLK = MPAD // MBLK


def _edge_body(xj_ref, c1_ref, b_ref, gmax_ref, gmin_ref, s_ref, sq_ref,
               acc_s, acc_q):
    pid = pl.program_id(0)

    @pl.when(pid == 0)
    def _():
        acc_s[...] = jnp.zeros((8, 128), jnp.float32)
        acc_q[...] = jnp.zeros((8, 128), jnp.float32)

    bvec = b_ref[...]

    def body(r, _):
        h = jnp.maximum(
            xj_ref[pl.ds(r * K, K), :] + c1_ref[pl.ds(r, 1), :] + bvec, 0.0)
        gmax_ref[pl.ds(r, 1), :] = jnp.max(h, axis=0, keepdims=True)
        gmin_ref[pl.ds(r, 1), :] = jnp.min(h, axis=0, keepdims=True)
        valid = (pid * MBLK + r) < M
        s1 = jnp.sum(h, axis=0, keepdims=True)
        q1 = jnp.sum(h * h, axis=0, keepdims=True)
        zero = jnp.zeros((1, 128), jnp.float32)
        acc_s[0:1, :] += jnp.where(valid, s1, zero)
        acc_q[0:1, :] += jnp.where(valid, q1, zero)
        return 0

    lax.fori_loop(0, MBLK, body, 0)

    @pl.when(pid == NEBLK - 1)
    def _():
        s_ref[...] = acc_s[...]
        sq_ref[...] = acc_q[...]


def _run_edge(xjb, subc1, brow):
    return pl.pallas_call(
        _edge_body,
        grid=(NEBLK,),
        in_specs=[
            pl.BlockSpec((MBLK * K, COUT), lambda i: (i, 0)),
            pl.BlockSpec((MBLK, COUT), lambda i: (i, 0)),
            pl.BlockSpec((1, COUT), lambda i: (0, 0)),
        ],
        out_specs=[
            pl.BlockSpec((MBLK, COUT), lambda i: (i, 0)),
            pl.BlockSpec((MBLK, COUT), lambda i: (i, 0)),
            pl.BlockSpec((8, COUT), lambda i: (0, 0)),
            pl.BlockSpec((8, COUT), lambda i: (0, 0)),
        ],
        out_shape=[
            jax.ShapeDtypeStruct((MPAD, COUT), jnp.float32),
            jax.ShapeDtypeStruct((MPAD, COUT), jnp.float32),
            jax.ShapeDtypeStruct((8, COUT), jnp.float32),
            jax.ShapeDtypeStruct((8, COUT), jnp.float32),
        ],
        scratch_shapes=[
            pltpu.VMEM((8, 128), jnp.float32),
            pltpu.VMEM((8, 128), jnp.float32),
        ],
    )(xjb, subc1, brow)


def _norm_body(gmax_ref, gmin_ref, s_ref, sq_ref, gamma_ref, beta_ref,
               out_ref):
    cnt = jnp.float32(M * K)
    mean = s_ref[0:1, :] / cnt
    var = sq_ref[0:1, :] / cnt - mean * mean
    sd = jnp.sqrt(var + jnp.float32(1e-5))
    gamma = gamma_ref[...]
    beta = beta_ref[...]
    g = jnp.where(gamma >= 0, gmax_ref[...], gmin_ref[...])
    out_ref[...] = (g - mean) / sd * gamma + beta


def _run_norm(gmax, gmin, s, sq, gamma, beta):
    return pl.pallas_call(
        _norm_body,
        out_shape=jax.ShapeDtypeStruct((MPAD, COUT), jnp.float32),
    )(gmax, gmin, s, sq, gamma, beta)


def kernel(x, pos, W, b, gamma, beta):
    p = pos[0, :, :, 0]
    xf = x[0, :, :, 0].T

    pad = NPAD - N
    pxr = jnp.pad(p[0:1], ((0, 0), (0, pad)), constant_values=PADV)
    pyr = jnp.pad(p[1:2], ((0, 0), (0, pad)), constant_values=PADV)
    pzr = jnp.pad(p[2:3], ((0, 0), (0, pad)), constant_values=PADV)
    px = pxr.reshape(NROWS, 128)
    py = pyr.reshape(NROWS, 128)
    pz = pzr.reshape(NROWS, 128)
    pt = jnp.pad(p.T, ((0, pad), (0, 5)), constant_values=0.0)
    pt = jnp.where(
        (jnp.arange(NPAD)[:, None] < N) | (jnp.arange(8)[None, :] >= 3),
        pt, PADV)

    fpsidx, subpos = _run_fps(px, py, pz, pt)
    nnidx = _run_knn(subpos.reshape(MPAD, 1, 8), pxr.reshape(1, 128, NGRP),
                     pyr.reshape(1, 128, NGRP), pzr.reshape(1, 128, NGRP))

    xfp = jnp.pad(xf, ((0, pad), (0, 0)))
    bmat, c1mat = _run_mm(xfp, W)

    nn_flat = nnidx[:, :K].reshape(-1)
    fps_flat = fpsidx[:, 0]
    xjb, subc1 = _run_sc_gather(bmat, c1mat, nn_flat, fps_flat)

    gmax, gmin, s, sq = _run_edge(xjb, subc1, b.reshape(1, COUT))
    out = _run_norm(gmax, gmin, s, sq, gamma.reshape(1, COUT),
                    beta.reshape(1, COUT))
    return out[:M].T[None, :, :, None]
